# baseline (speedup 1.0000x reference)
import jax
import jax.numpy as jnp
import numpy as np
from jax.experimental import pallas as pl
from jax.experimental.pallas import tpu as pltpu

N = 256
H = 256
B = 8
ATT_H = 128
L = 3
_NPAIR = (N - 2) * (N - 1) // 2
_NZERO = N * N - _NPAIR
_LN2 = float(np.log(2.0))


def _shift_up(x):
    return jnp.concatenate([x[1:, :], jnp.zeros((1, x.shape[1]), jnp.float32)], axis=0)


def _body(rew_ref, adj_ref, *refs):
    nparams = 12 * L + 6
    prefs = refs[:nparams]
    out_ref = refs[nparams]
    G_ref = refs[nparams + 1]
    Wm_ref = refs[nparams + 2]

    row = jax.lax.broadcasted_iota(jnp.int32, (N, N), 0)
    col = jax.lax.broadcasted_iota(jnp.int32, (N, N), 1)
    state = jnp.where(row == col, 1.0, 0.0).astype(jnp.float32)

    for l in range(L):
        (msg_w1, msg_b1, msg_w2, msg_b2,
         att_w1, att_b1, att_w2, att_b2,
         gru_wih, gru_bih, gru_whh, gru_bhh) = (r[...] for r in prefs[12 * l:12 * l + 12])
        msg_b1, msg_b2, att_b1, att_b2, gru_bih, gru_bhh = (
            v.reshape(1, -1) for v in (msg_b1, msg_b2, att_b1, att_b2, gru_bih, gru_bhh))

        wd_m, wu_m = msg_w1[:H, :], msg_w1[H:, :]
        wd_a, wu_a = att_w1[:H, :], att_w1[H:, :]

        stb = state.astype(jnp.bfloat16)

        sw = jnp.dot(stb, wd_m.astype(jnp.bfloat16), preferred_element_type=jnp.float32)
        msg_pre = (sw - _shift_up(sw)) + (_shift_up(wu_m) - wu_m) + msg_b1
        msg = jnp.dot(jnp.maximum(msg_pre, 0.0).astype(jnp.bfloat16),
                      msg_w2.astype(jnp.bfloat16),
                      preferred_element_type=jnp.float32) + msg_b2

        sa = jnp.dot(stb, wd_a.astype(jnp.bfloat16), preferred_element_type=jnp.float32)
        att_pre = (sa - _shift_up(sa)) + (_shift_up(wu_a) - wu_a) + att_b1
        att = jax.nn.sigmoid(
            jnp.dot(jnp.maximum(att_pre, 0.0).astype(jnp.bfloat16),
                    att_w2.astype(jnp.bfloat16),
                    preferred_element_type=jnp.float32) + att_b2)

        m = msg * att
        state_msg = jnp.concatenate([jnp.zeros((1, H), jnp.float32), m[:N - 1, :]], axis=0)

        gi = jnp.dot(state_msg.astype(jnp.bfloat16), gru_wih.astype(jnp.bfloat16),
                     preferred_element_type=jnp.float32) + gru_bih
        gh = jnp.dot(stb, gru_whh.astype(jnp.bfloat16),
                     preferred_element_type=jnp.float32) + gru_bhh
        r_g = jax.nn.sigmoid(gi[:, :H] + gh[:, :H])
        z_g = jax.nn.sigmoid(gi[:, H:2 * H] + gh[:, H:2 * H])
        n_g = jnp.tanh(gi[:, 2 * H:] + r_g * gh[:, 2 * H:])
        state = (1.0 - z_g) * n_g + z_g * state

    ow1, ob1, ow2, ob2, ow3, ob3 = (r[...] for r in prefs[12 * L:])
    ob1 = ob1.reshape(1, -1)
    ob2 = ob2.reshape(1, -1)
    ob3 = ob3.reshape(1, 1)
    ow3r = ow3.T
    ow2b = ow2.astype(jnp.bfloat16)

    G_ref[...] = jnp.dot(state.astype(jnp.bfloat16), ow1.astype(jnp.bfloat16),
                         preferred_element_type=jnp.float32)

    wm = jnp.zeros((N, N), jnp.float32)
    for b in range(B):
        wm = wm + rew_ref[b] * adj_ref[b, 0]
    Wm_ref[...] = wm

    jj = jax.lax.broadcasted_iota(jnp.int32, (N, 1), 0)
    lane = jax.lax.broadcasted_iota(jnp.int32, (1, N), 1)

    def pair_step(i, carry):
        Gc, Tacc = carry
        q = 253 - i
        gi_row = G_ref[pl.ds(i, 1), :] + ob1
        gq_row = G_ref[pl.ds(q, 1), :] + ob1
        part1 = jj < 254 - i
        src = jnp.where(part1, Gc, G_ref[...])
        top = jnp.where(part1, gi_row, gq_row)
        h1 = jnp.maximum(top - src, 0.0).astype(jnp.bfloat16)
        h2 = jnp.maximum(jnp.dot(h1, ow2b, preferred_element_type=jnp.float32) + ob2, 0.0)
        traw = jnp.sum(h2 * ow3r, axis=1, keepdims=True)
        Tacc = jnp.where((lane == i) | (lane == q), traw, Tacc)
        Gc = pltpu.roll(Gc, N - 1, axis=0)
        return Gc, Tacc

    z = jnp.zeros((N, N), jnp.float32)
    Gc0 = pltpu.roll(G_ref[...], N - 2, axis=0)
    _, Tacc = jax.lax.fori_loop(0, 127, pair_step, (Gc0, z), unroll=32)

    TaccT = Tacc.T
    shifts = jnp.where(jj <= 126, jj + 2, 0)
    TunT = TaccT
    for k in range(8):
        rolled = pltpu.roll(TunT, 1 << k, axis=1)
        TunT = jnp.where((shifts & (1 << k)) != 0, rolled, TunT)
    maskT = lane >= jj + 2
    T = jnp.where(maskT, TunT + ob3, 0.0)

    sp_all = jnp.sum(jnp.maximum(T, 0.0) + jnp.log1p(jnp.exp(-jnp.abs(T))))
    acc2 = jnp.sum(T * Wm_ref[...].T)

    s_rew = rew_ref[0]
    for b in range(1, B):
        s_rew = s_rew + rew_ref[b]

    loss = (s_rew * sp_all - acc2) * (1.0 / (N * N * B))
    out_ref[0, 0] = loss


def kernel(adj, rewards, params):
    flat = []
    for lp in params['layers']:
        flat += [lp['msg_w1'], lp['msg_b1'],
                 lp['msg_w2'], lp['msg_b2'],
                 lp['att_w1'], lp['att_b1'],
                 lp['att_w2'], lp['att_b2'],
                 lp['gru_wih'], lp['gru_bih'],
                 lp['gru_whh'], lp['gru_bhh']]
    flat += [params['out_w1'], params['out_b1'],
             params['out_w2'], params['out_b2'],
             params['out_w3'], params['out_b3']]

    in_specs = ([pl.BlockSpec(memory_space=pltpu.SMEM),
                 pl.BlockSpec(memory_space=pltpu.VMEM)] +
                [pl.BlockSpec(memory_space=pltpu.VMEM)] * len(flat))

    out = pl.pallas_call(
        _body,
        out_shape=jax.ShapeDtypeStruct((1, 1), jnp.float32),
        in_specs=in_specs,
        out_specs=pl.BlockSpec(memory_space=pltpu.SMEM),
        scratch_shapes=[pltpu.VMEM((N, N), jnp.float32),
                        pltpu.VMEM((N, N), jnp.float32)],
    )(rewards, adj, *flat)
    return out[0, 0]


# speedup vs baseline: 1.0075x; 1.0075x over previous
import jax
import jax.numpy as jnp
import numpy as np
from jax.experimental import pallas as pl
from jax.experimental.pallas import tpu as pltpu

N = 256
H = 256
B = 8
ATT_H = 128
L = 3
_NPAIR = (N - 2) * (N - 1) // 2
_NZERO = N * N - _NPAIR
_LN2 = float(np.log(2.0))


def _shift_up(x):
    return jnp.concatenate([x[1:, :], jnp.zeros((1, x.shape[1]), jnp.float32)], axis=0)


def _body(rew_ref, adj_ref, *refs):
    nparams = 12 * L + 6
    prefs = refs[:nparams]
    out_ref = refs[nparams]
    G_ref = refs[nparams + 1]
    Wm_ref = refs[nparams + 2]

    row = jax.lax.broadcasted_iota(jnp.int32, (N, N), 0)
    col = jax.lax.broadcasted_iota(jnp.int32, (N, N), 1)
    state = jnp.where(row == col, 1.0, 0.0).astype(jnp.float32)

    for l in range(L):
        (msg_w1, msg_b1, msg_w2, msg_b2,
         att_w1, att_b1, att_w2, att_b2,
         gru_wih, gru_bih, gru_whh, gru_bhh) = (r[...] for r in prefs[12 * l:12 * l + 12])
        msg_b1, msg_b2, att_b1, att_b2, gru_bih, gru_bhh = (
            v.reshape(1, -1) for v in (msg_b1, msg_b2, att_b1, att_b2, gru_bih, gru_bhh))

        wd_m, wu_m = msg_w1[:H, :], msg_w1[H:, :]
        wd_a, wu_a = att_w1[:H, :], att_w1[H:, :]

        sw = jnp.dot(state, wd_m, preferred_element_type=jnp.float32)
        msg_pre = (sw - _shift_up(sw)) + (_shift_up(wu_m) - wu_m) + msg_b1
        msg = jnp.dot(jnp.maximum(msg_pre, 0.0), msg_w2,
                      preferred_element_type=jnp.float32) + msg_b2

        sa = jnp.dot(state, wd_a, preferred_element_type=jnp.float32)
        att_pre = (sa - _shift_up(sa)) + (_shift_up(wu_a) - wu_a) + att_b1
        att = jax.nn.sigmoid(
            jnp.dot(jnp.maximum(att_pre, 0.0), att_w2,
                    preferred_element_type=jnp.float32) + att_b2)

        m = msg * att
        state_msg = jnp.concatenate([jnp.zeros((1, H), jnp.float32), m[:N - 1, :]], axis=0)

        gi = jnp.dot(state_msg, gru_wih, preferred_element_type=jnp.float32) + gru_bih
        gh = jnp.dot(state, gru_whh, preferred_element_type=jnp.float32) + gru_bhh
        r_g = jax.nn.sigmoid(gi[:, :H] + gh[:, :H])
        z_g = jax.nn.sigmoid(gi[:, H:2 * H] + gh[:, H:2 * H])
        n_g = jnp.tanh(gi[:, 2 * H:] + r_g * gh[:, 2 * H:])
        state = (1.0 - z_g) * n_g + z_g * state

    ow1, ob1, ow2, ob2, ow3, ob3 = (r[...] for r in prefs[12 * L:])
    ob1 = ob1.reshape(1, -1)
    ob2 = ob2.reshape(1, -1)
    ob3 = ob3.reshape(1, 1)
    ow3r = ow3.T
    ow2b = ow2.astype(jnp.bfloat16)

    G_ref[...] = jnp.dot(state, ow1, preferred_element_type=jnp.float32)

    wm = jnp.zeros((N, N), jnp.float32)
    for b in range(B):
        wm = wm + rew_ref[b] * adj_ref[b, 0]
    Wm_ref[...] = wm

    jj = jax.lax.broadcasted_iota(jnp.int32, (N, 1), 0)
    lane = jax.lax.broadcasted_iota(jnp.int32, (1, N), 1)

    def pair_step(i, carry):
        Gc, Tacc = carry
        q = 253 - i
        gi_row = G_ref[pl.ds(i, 1), :] + ob1
        gq_row = G_ref[pl.ds(q, 1), :] + ob1
        part1 = jj < 254 - i
        src = jnp.where(part1, Gc, G_ref[...])
        top = jnp.where(part1, gi_row, gq_row)
        h1 = jnp.maximum(top - src, 0.0).astype(jnp.bfloat16)
        h2 = jnp.maximum(jnp.dot(h1, ow2b, preferred_element_type=jnp.float32) + ob2, 0.0)
        traw = jnp.sum(h2 * ow3r, axis=1, keepdims=True)
        Tacc = jnp.where((lane == i) | (lane == q), traw, Tacc)
        Gc = pltpu.roll(Gc, N - 1, axis=0)
        return Gc, Tacc

    z = jnp.zeros((N, N), jnp.float32)
    Gc0 = pltpu.roll(G_ref[...], N - 2, axis=0)
    _, Tacc = jax.lax.fori_loop(0, 127, pair_step, (Gc0, z), unroll=32)

    TaccT = Tacc.T
    shifts = jnp.where(jj <= 126, jj + 2, 0)
    TunT = TaccT
    for k in range(8):
        rolled = pltpu.roll(TunT, 1 << k, axis=1)
        TunT = jnp.where((shifts & (1 << k)) != 0, rolled, TunT)
    maskT = lane >= jj + 2
    T = jnp.where(maskT, TunT + ob3, 0.0)

    sp_all = jnp.sum(jnp.maximum(T, 0.0) + jnp.log1p(jnp.exp(-jnp.abs(T))))
    acc2 = jnp.sum(T * Wm_ref[...].T)

    s_rew = rew_ref[0]
    for b in range(1, B):
        s_rew = s_rew + rew_ref[b]

    loss = (s_rew * sp_all - acc2) * (1.0 / (N * N * B))
    out_ref[0, 0] = loss


def kernel(adj, rewards, params):
    flat = []
    for lp in params['layers']:
        flat += [lp['msg_w1'], lp['msg_b1'],
                 lp['msg_w2'], lp['msg_b2'],
                 lp['att_w1'], lp['att_b1'],
                 lp['att_w2'], lp['att_b2'],
                 lp['gru_wih'], lp['gru_bih'],
                 lp['gru_whh'], lp['gru_bhh']]
    flat += [params['out_w1'], params['out_b1'],
             params['out_w2'], params['out_b2'],
             params['out_w3'], params['out_b3']]

    in_specs = ([pl.BlockSpec(memory_space=pltpu.SMEM),
                 pl.BlockSpec(memory_space=pltpu.VMEM)] +
                [pl.BlockSpec(memory_space=pltpu.VMEM)] * len(flat))

    out = pl.pallas_call(
        _body,
        out_shape=jax.ShapeDtypeStruct((1, 1), jnp.float32),
        in_specs=in_specs,
        out_specs=pl.BlockSpec(memory_space=pltpu.SMEM),
        scratch_shapes=[pltpu.VMEM((N, N), jnp.float32),
                        pltpu.VMEM((N, N), jnp.float32)],
    )(rewards, adj, *flat)
    return out[0, 0]
